# baseline (speedup 1.0000x reference)
import functools

import jax
import jax.numpy as jnp
from jax.experimental import pallas as pl

N = 50000
E = 800000
G = 16
H1 = 16
HID = 128
NEG_SLOPE = 0.01
BN_EPS = 1e-5

NB = 1000


def _mlp_block_kernel(z_ref, w1_ref, b1_ref, w2_ref, b2_ref, o_ref):
    z = z_ref[...]
    if z.shape[1] == 1:
        h = jnp.maximum(z * w1_ref[...] + b1_ref[...], 0.0)
    else:
        h = jnp.maximum(jnp.dot(z, w1_ref[...], preferred_element_type=jnp.float32)
                        + b1_ref[...], 0.0)
    h = jnp.maximum(jnp.dot(h, w2_ref[...], preferred_element_type=jnp.float32)
                    + b2_ref[...], 0.0)
    o_ref[...] = h


@functools.partial(jax.jit, static_argnames=("fi",))
def _mlp(z, w1, b1, w2, b2, fi):
    grid = (N // NB,)
    return pl.pallas_call(
        _mlp_block_kernel,
        grid=grid,
        in_specs=[
            pl.BlockSpec((NB, fi), lambda i: (i, 0)),
            pl.BlockSpec((fi, HID), lambda i: (0, 0)),
            pl.BlockSpec((1, HID), lambda i: (0, 0)),
            pl.BlockSpec((HID, HID), lambda i: (0, 0)),
            pl.BlockSpec((1, HID), lambda i: (0, 0)),
        ],
        out_specs=pl.BlockSpec((NB, HID), lambda i: (i, 0)),
        out_shape=jax.ShapeDtypeStruct((N, HID), jnp.float32),
    )(z, w1, b1, w2, b2)


def _bn(h, gamma, beta):
    mu = h.mean(axis=0)
    var = h.var(axis=0)
    return (h - mu) / jnp.sqrt(var + BN_EPS) * gamma + beta


def _gin(h, row, col, eps, mlp):
    agg = jax.ops.segment_sum(h[row], col, num_segments=N)
    z = (1.0 + eps) * h + agg
    w1, b1, w2, b2, g, bt = mlp
    fi = z.shape[1]
    z = _mlp(z, w1, b1[None, :], w2, b2[None, :], fi)
    return _bn(z, g, bt)


def _get_mask(m, row, col):
    mm = (jnp.abs(m).sum(axis=-1, keepdims=True) > 0).astype(jnp.float32)
    agg = jax.ops.segment_max(mm[row], col, num_segments=N)
    agg = jnp.where(jnp.isfinite(agg), agg, 0.0)
    return jnp.maximum(mm, agg)


def _gnorm(h, batch):
    cnt = jax.ops.segment_sum(jnp.ones((h.shape[0],), jnp.float32), batch, num_segments=G)
    inv = 1.0 / jnp.sqrt(jnp.maximum(cnt, 1.0))
    return h * inv[batch][:, None]


def kernel(x, edge_index, batch, params):
    row, col = edge_index[0], edge_index[1]
    h = x[:, None]
    mask = _get_mask(h, row, col)
    h = jax.nn.leaky_relu(_gin(h, row, col, params['eps1'], params['mlp1']), NEG_SLOPE)
    h = h * mask
    h = _gnorm(h, batch)
    h = _bn(h, params['bn1'][0], params['bn1'][1])
    for conv, bn in zip(params['convs'], params['bns']):
        h = h + jax.nn.leaky_relu(_gin(h, row, col, conv['eps'], conv['mlp']), NEG_SLOPE)
        mask = _get_mask(mask, row, col)
        h = h * mask
        h = _gnorm(h, batch)
        h = _bn(h, bn[0], bn[1])
    w1, b1 = params['lin1']
    h = jax.nn.leaky_relu(h @ w1 + b1, NEG_SLOPE)
    h = h * mask
    w2, b2 = params['lin2']
    h = jax.nn.leaky_relu(h @ w2 + b2, NEG_SLOPE)
    h = h * mask
    bmax = jax.ops.segment_max(h, batch, num_segments=G)
    bmax = jnp.where(jnp.isfinite(bmax), bmax, 0.0)[batch]
    bmin = jax.ops.segment_min(h, batch, num_segments=G)
    bmin = jnp.where(jnp.isfinite(bmin), bmin, 0.0)[batch]
    probs = (h - bmin) / (bmax + 1e-06 - bmin)
    deg = jnp.bincount(row, length=N).astype(jnp.float32)[:, None]
    t1 = jax.ops.segment_sum(probs * deg, batch, num_segments=G)
    t2 = jax.ops.segment_sum(probs[row] * probs[col], batch[row], num_segments=G)
    loss = -(t1 - t2)
    return probs[:, 0], loss[:, 0], loss.mean()


# speedup vs baseline: 5.7707x; 5.7707x over previous
import functools

import jax
import jax.numpy as jnp
from jax import lax
from jax.experimental import pallas as pl
from jax.experimental.pallas import tpu as pltpu
from jax.experimental.pallas import tpu_sc as plsc

N = 50000
E = 800000
G = 16
H1 = 16
HID = 128
NEG_SLOPE = 0.01
BN_EPS = 1e-5

NB = 1000

NC = 2
NS = 16
NW = NC * NS
EPW = 25088
EPAD = NW * EPW
CH = 3136
NCHUNK = EPW // CH
NSL = 3136
NPAD = NSL * NS
RW = 4


def _edge_gs_body(x_hbm, a_hbm, b_hbm, out_hbm,
                  x_v, a_v, b_v, tmp_v, red_v, acc_v, stage):
    c = lax.axis_index("c")
    s = lax.axis_index("s")
    base = (c * NS + s) * EPW

    pltpu.sync_copy(x_hbm, x_v)

    @pl.loop(0, NPAD // 16)
    def _z(i):
        acc_v[pl.ds(i * 16, 16)] = jnp.zeros((16,), jnp.float32)

    @pl.loop(0, NCHUNK)
    def _chunk(j):
        off = base + j * CH
        pltpu.sync_copy(a_hbm.at[pl.ds(off, CH)], a_v)
        pltpu.sync_copy(b_hbm.at[pl.ds(off, CH)], b_v)

        @pl.loop(0, CH // 16)
        def _gs(i):
            av = a_v[pl.ds(i * 16, 16)]
            bv = b_v[pl.ds(i * 16, 16)]
            vals = plsc.load_gather(x_v, [av])
            plsc.addupdate_scatter(acc_v, [bv], vals)

    @pl.loop(0, NSL // 16)
    def _zr(i):
        red_v[pl.ds(i * 16, 16)] = jnp.zeros((16,), jnp.float32)

    for w in range(NS // RW):
        @pl.when(jnp.logical_and(s >= w * RW, s < (w + 1) * RW))
        def _pub():
            pltpu.sync_copy(acc_v, stage.at[pl.ds((s - w * RW) * NPAD, NPAD)])
        plsc.subcore_barrier()

        @pl.loop(0, RW)
        def _red(t):
            pltpu.sync_copy(stage.at[pl.ds(t * NPAD + s * NSL, NSL)], tmp_v)

            @pl.loop(0, NSL // 16)
            def _acc(i):
                red_v[pl.ds(i * 16, 16)] = red_v[pl.ds(i * 16, 16)] + tmp_v[pl.ds(i * 16, 16)]

        plsc.subcore_barrier()

    pltpu.sync_copy(red_v, out_hbm.at[pl.ds(c * NPAD + s * NSL, NSL)])


def _edge_gs(x, a, b):
    mesh = plsc.VectorSubcoreMesh(core_axis_name="c", subcore_axis_name="s")
    return pl.kernel(
        _edge_gs_body,
        mesh=mesh,
        compiler_params=pltpu.CompilerParams(needs_layout_passes=False),
        out_type=jax.ShapeDtypeStruct((NC * NPAD,), jnp.float32),
        scratch_types=[
            pltpu.VMEM((N,), jnp.float32),
            pltpu.VMEM((CH,), jnp.int32),
            pltpu.VMEM((CH,), jnp.int32),
            pltpu.VMEM((NSL,), jnp.float32),
            pltpu.VMEM((NSL,), jnp.float32),
            pltpu.VMEM((NPAD,), jnp.float32),
            pltpu.VMEM_SHARED((RW * NPAD,), jnp.float32),
        ],
    )(x, a, b)


def _mlp_block_kernel(z_ref, w1_ref, b1_ref, w2_ref, b2_ref, o_ref):
    z = z_ref[...]
    if z.shape[1] == 1:
        h = jnp.maximum(z * w1_ref[...] + b1_ref[...], 0.0)
    else:
        h = jnp.maximum(jnp.dot(z, w1_ref[...], preferred_element_type=jnp.float32)
                        + b1_ref[...], 0.0)
    h = jnp.maximum(jnp.dot(h, w2_ref[...], preferred_element_type=jnp.float32)
                    + b2_ref[...], 0.0)
    o_ref[...] = h


@functools.partial(jax.jit, static_argnames=("fi",))
def _mlp(z, w1, b1, w2, b2, fi):
    grid = (N // NB,)
    return pl.pallas_call(
        _mlp_block_kernel,
        grid=grid,
        in_specs=[
            pl.BlockSpec((NB, fi), lambda i: (i, 0)),
            pl.BlockSpec((fi, HID), lambda i: (0, 0)),
            pl.BlockSpec((1, HID), lambda i: (0, 0)),
            pl.BlockSpec((HID, HID), lambda i: (0, 0)),
            pl.BlockSpec((1, HID), lambda i: (0, 0)),
        ],
        out_specs=pl.BlockSpec((NB, HID), lambda i: (i, 0)),
        out_shape=jax.ShapeDtypeStruct((N, HID), jnp.float32),
    )(z, w1, b1, w2, b2)


def _bn(h, gamma, beta):
    mu = h.mean(axis=0)
    var = h.var(axis=0)
    return (h - mu) / jnp.sqrt(var + BN_EPS) * gamma + beta


def _gin(h, row, col, eps, mlp, agg=None):
    if agg is None:
        agg = jax.ops.segment_sum(h[row], col, num_segments=N)
    z = (1.0 + eps) * h + agg
    w1, b1, w2, b2, g, bt = mlp
    fi = z.shape[1]
    z = _mlp(z, w1, b1[None, :], w2, b2[None, :], fi)
    return _bn(z, g, bt)


def _gnorm(h, batch):
    cnt = jax.ops.segment_sum(jnp.ones((h.shape[0],), jnp.float32), batch, num_segments=G)
    inv = 1.0 / jnp.sqrt(jnp.maximum(cnt, 1.0))
    return h * inv[batch][:, None]


def kernel(x, edge_index, batch, params):
    row, col = edge_index[0], edge_index[1]

    pad0 = jnp.zeros((EPAD - E,), jnp.int32)
    padN = jnp.full((EPAD - E,), N, jnp.int32)
    rowA = jnp.concatenate([row, pad0])
    colA = jnp.concatenate([col, pad0])
    rowB = jnp.concatenate([row, padN])
    colB = jnp.concatenate([col, padN])

    ones = jnp.ones((N,), jnp.float32)
    deg_p = _edge_gs(ones, rowA, rowB)
    deg = (deg_p[:N] + deg_p[NPAD:NPAD + N])[:, None]
    agg1_p = _edge_gs(x, rowA, colB)
    agg1 = (agg1_p[:N] + agg1_p[NPAD:NPAD + N])[:, None]
    mm0 = (x != 0.0).astype(jnp.float32)
    msum_p = _edge_gs(mm0, rowA, colB)
    mask = jnp.maximum(mm0, ((msum_p[:N] + msum_p[NPAD:NPAD + N]) > 0.0).astype(jnp.float32))[:, None]

    h = x[:, None]
    h = jax.nn.leaky_relu(_gin(h, row, col, params['eps1'], params['mlp1'], agg=agg1),
                          NEG_SLOPE)
    h = h * mask
    h = _gnorm(h, batch)
    h = _bn(h, params['bn1'][0], params['bn1'][1])
    for conv, bn in zip(params['convs'], params['bns']):
        h = h + jax.nn.leaky_relu(_gin(h, row, col, conv['eps'], conv['mlp']), NEG_SLOPE)
        msk_p = _edge_gs(mask[:, 0], rowA, colB)
        mask = jnp.maximum(mask, ((msk_p[:N] + msk_p[NPAD:NPAD + N]) > 0.0).astype(jnp.float32)[:, None])
        h = h * mask
        h = _gnorm(h, batch)
        h = _bn(h, bn[0], bn[1])
    w1, b1 = params['lin1']
    h = jax.nn.leaky_relu(h @ w1 + b1, NEG_SLOPE)
    h = h * mask
    w2, b2 = params['lin2']
    h = jax.nn.leaky_relu(h @ w2 + b2, NEG_SLOPE)
    h = h * mask
    bmax = jax.ops.segment_max(h, batch, num_segments=G)
    bmax = jnp.where(jnp.isfinite(bmax), bmax, 0.0)[batch]
    bmin = jax.ops.segment_min(h, batch, num_segments=G)
    bmin = jnp.where(jnp.isfinite(bmin), bmin, 0.0)[batch]
    probs = (h - bmin) / (bmax + 1e-06 - bmin)
    s_p = _edge_gs(probs[:, 0], colA, rowB)
    s = (s_p[:N] + s_p[NPAD:NPAD + N])[:, None]
    t1 = jax.ops.segment_sum(probs * deg, batch, num_segments=G)
    t2 = jax.ops.segment_sum(probs * s, batch, num_segments=G)
    loss = -(t1 - t2)
    return probs[:, 0], loss[:, 0], loss.mean()
